# baseline (speedup 1.0000x reference)
import functools

import jax
import jax.numpy as jnp
from jax import lax
from jax.experimental import pallas as pl
from jax.experimental.pallas import tpu as pltpu
from jax.experimental.pallas import tpu_sc as plsc

N = 10000
E = 320000
D = 128
DE = 16
ENF = 64

_INFO = plsc.get_sparse_core_info()
NC = _INFO.num_cores
NS = _INFO.num_subcores
NW = NC * NS
EPW = E // NW
CH = 80
HCH = CH // 2
NITER = EPW // CH
RPT = 624
RPT_LAST = N - 15 * RPT

_mesh = plsc.VectorSubcoreMesh(core_axis_name="c", subcore_axis_name="s")

_PERM = [32 * k + 2 * j + r for k in range(2) for r in range(2)
         for j in range(16)]


def _proj_body(h_ref, w_ref, hst_ref):
    hst_ref[...] = jnp.dot(h_ref[...], w_ref[...],
                           preferred_element_type=jnp.float32)


def _project(h, wst):
    return pl.pallas_call(
        _proj_body,
        out_shape=jax.ShapeDtypeStruct((N, D), jnp.float32),
    )(h, wst)


@functools.partial(
    pl.kernel,
    mesh=_mesh,
    out_type=jax.ShapeDtypeStruct((E // 2, D), jnp.float32),
    scratch_types=[
        pltpu.VMEM((NITER, CH), jnp.int32),
        pltpu.VMEM((NITER, CH), jnp.int32),
        pltpu.VMEM((3, CH, D), jnp.float32),
        pltpu.VMEM((3, CH, D), jnp.float32),
        pltpu.VMEM((2, HCH, D), jnp.float32),
        pltpu.SemaphoreType.DMA((3,)),
        pltpu.SemaphoreType.DMA((3,)),
        pltpu.SemaphoreType.DMA((2,)),
    ],
)
def _gather_k(hst_hbm, row_hbm, col_hbm, u_hbm,
              idxr, idxc, g1, g2, ub, sem1, sem2, semu):
    wid = lax.axis_index("s") * NC + lax.axis_index("c")
    pltpu.sync_copy(row_hbm.at[wid], idxr)
    pltpu.sync_copy(col_hbm.at[wid], idxc)
    ubase = wid * (EPW // 2)

    def start(i, s):
        pltpu.async_copy(hst_hbm.at[idxr.at[i]], g1.at[s], sem1.at[s])
        pltpu.async_copy(hst_hbm.at[idxc.at[i]], g2.at[s], sem2.at[s])

    def wait(s):
        pltpu.make_async_copy(hst_hbm.at[pl.ds(0, CH)], g1.at[s],
                              sem1.at[s]).wait()
        pltpu.make_async_copy(hst_hbm.at[pl.ds(0, CH)], g2.at[s],
                              sem2.at[s]).wait()

    start(0, 0)
    start(1, 1)

    def body(i, carry):
        s = lax.rem(i, 3)
        su = i & 1

        @pl.when(i + 2 < NITER)
        def _():
            start(i + 2, lax.rem(i + 2, 3))

        wait(s)

        @pl.when(i >= 2)
        def _():
            pltpu.make_async_copy(u_hbm.at[pl.ds(0, HCH)], ub.at[su],
                                  semu.at[su]).wait()

        def cbody(p, c2):
            e0 = 2 * p
            e1 = e0 + 1
            for k in range(4):
                lo = 16 * k
                hi = ENF + lo
                ub[su, p, pl.ds(lo, 16)] = (g1[s, e0, pl.ds(lo, 16)]
                                            + g2[s, e0, pl.ds(hi, 16)])
                ub[su, p, pl.ds(hi, 16)] = (g1[s, e1, pl.ds(lo, 16)]
                                            + g2[s, e1, pl.ds(hi, 16)])
            return c2

        lax.fori_loop(0, HCH, cbody, 0)
        uoff = pl.multiple_of(ubase + i * HCH, HCH)
        pltpu.async_copy(ub.at[su], u_hbm.at[pl.ds(uoff, HCH)], semu.at[su])
        return carry

    lax.fori_loop(0, NITER, body, 0)
    pltpu.make_async_copy(u_hbm.at[pl.ds(0, HCH)], ub.at[0], semu.at[0]).wait()
    pltpu.make_async_copy(u_hbm.at[pl.ds(0, HCH)], ub.at[1], semu.at[1]).wait()


def _edge_mlp_body(u_ref, ea_ref, wa_ref, b1_ref, w2_ref, b2_ref, out_ref):
    u = u_ref[...]
    ea = ea_ref[...]

    def half(ux, eax):
        pre = (ux
               + jnp.dot(eax, wa_ref[...], preferred_element_type=jnp.float32)
               + b1_ref[...])
        t = jax.nn.silu(pre)
        return jax.nn.silu(
            jnp.dot(t, w2_ref[...], preferred_element_type=jnp.float32)
            + b2_ref[...])

    out_ref[...] = jnp.concatenate(
        [half(u[:, :ENF], ea[:, :DE]), half(u[:, ENF:], ea[:, DE:])], axis=1)


def _edge_mlp(u2, ea2, wa, b1, w2, b2):
    BEH = 2000
    grid = (E // 2 // BEH,)
    blk = lambda r, c: pl.BlockSpec((r, c), lambda i: (i, 0))
    fixed = lambda r, c: pl.BlockSpec((r, c), lambda i: (0, 0))
    return pl.pallas_call(
        _edge_mlp_body,
        grid=grid,
        in_specs=[blk(BEH, D), blk(BEH, 2 * DE),
                  fixed(DE, ENF), fixed(1, ENF), fixed(ENF, ENF),
                  fixed(1, ENF)],
        out_specs=blk(BEH, D),
        out_shape=jax.ShapeDtypeStruct((E // 2, D), jnp.float32),
    )(u2, ea2, wa, b1, w2, b2)


@functools.partial(
    pl.kernel,
    mesh=_mesh,
    out_type=jax.ShapeDtypeStruct((NC, N, D), jnp.float32),
    scratch_types=[
        pltpu.VMEM((NITER, CH), jnp.int32),
        pltpu.VMEM((2, CH, D), jnp.float32),
        pltpu.VMEM_SHARED((N, D), jnp.float32),
        pltpu.SemaphoreType.DMA((2,)),
    ],
)
def _scatter_k(ef_hbm, row_hbm, zero_hbm, out_hbm, idxd, buf, acc, sem):
    cid = lax.axis_index("c")
    sid = lax.axis_index("s")
    start = pl.multiple_of(sid * RPT, 8)
    @pl.when(sid < NS - 1)
    def _():
        pltpu.sync_copy(zero_hbm.at[pl.ds(start, RPT)],
                        acc.at[pl.ds(start, RPT)])

    @pl.when(sid == NS - 1)
    def _():
        pltpu.sync_copy(zero_hbm.at[pl.ds(start, RPT_LAST)],
                        acc.at[pl.ds(start, RPT_LAST)])

    pltpu.sync_copy(row_hbm.at[cid, sid], idxd)
    plsc.subcore_barrier()
    base = sid * (E // 2 // NS)

    def load(i, s):
        off = pl.multiple_of(base + i * CH, CH)
        pltpu.async_copy(ef_hbm.at[pl.ds(off, CH)], buf.at[s], sem.at[s])

    def wait(s):
        pltpu.make_async_copy(ef_hbm.at[pl.ds(0, CH)], buf.at[s],
                              sem.at[s]).wait()

    load(0, 0)

    def body(i, carry):
        s = i & 1

        @pl.when(i + 1 < NITER)
        def _():
            load(i + 1, 1 - s)

        wait(s)
        pltpu.sync_copy(buf.at[s], acc.at[idxd.at[i]], add=True)
        return carry

    lax.fori_loop(0, NITER, body, 0)
    plsc.subcore_barrier()

    @pl.when(sid < NS - 1)
    def _():
        pltpu.sync_copy(acc.at[pl.ds(start, RPT)],
                        out_hbm.at[cid, pl.ds(start, RPT)])

    @pl.when(sid == NS - 1)
    def _():
        pltpu.sync_copy(acc.at[pl.ds(start, RPT_LAST)],
                        out_hbm.at[cid, pl.ds(start, RPT_LAST)])


def _node_mlp_body(h_ref, a0_ref, a1_ref, wh_ref, wa_ref, b1_ref, w2_ref,
                   b2_ref, out_ref):
    hcur = h_ref[...]
    agg = a0_ref[:, :ENF] + a1_ref[:, ENF:]
    z = jax.nn.silu(
        jnp.dot(hcur, wh_ref[...], preferred_element_type=jnp.float32)
        + jnp.dot(agg, wa_ref[...], preferred_element_type=jnp.float32)
        + b1_ref[...])
    out_ref[...] = (hcur
                    + jnp.dot(z, w2_ref[...],
                              preferred_element_type=jnp.float32)
                    + b2_ref[...])


def _node_mlp(h, a0, a1, wh, wa, b1, w2, b2):
    return pl.pallas_call(
        _node_mlp_body,
        out_shape=jax.ShapeDtypeStruct((N, D), jnp.float32),
    )(h, a0, a1, wh, wa, b1, w2, b2)


def kernel(h, edge_index, edge_attr, We1, be1, We2, be2, Wn1, bn1, Wn2, bn2):
    row = edge_index[0].astype(jnp.int32)
    col = edge_index[1].astype(jnp.int32)
    wst = jnp.concatenate([We1[:D], We1[D:2 * D]], axis=1)
    hst = _project(h, wst)
    row3 = row.reshape(NW, NITER, CH)
    col3 = col.reshape(NW, NITER, CH)
    u2 = _gather_k(hst, row3, col3)
    ea2 = edge_attr.reshape(E // 2, 2 * DE)
    ef = _edge_mlp(u2, ea2, We1[2 * D:], be1.reshape(1, ENF), We2,
                   be2.reshape(1, ENF))
    rowpair = row.reshape(E // 2, 2)
    idxeo = jnp.stack([rowpair[:, 0].reshape(NS, NITER, CH),
                       rowpair[:, 1].reshape(NS, NITER, CH)])
    parts = _scatter_k(ef, idxeo, jnp.zeros((N, D), jnp.float32))
    return _node_mlp(h, parts[0], parts[1], Wn1[:D], Wn1[D:],
                     bn1.reshape(1, D), Wn2, bn2.reshape(1, D))


# speedup vs baseline: 1.0441x; 1.0441x over previous
import functools

import jax
import jax.numpy as jnp
from jax import lax
from jax.experimental import pallas as pl
from jax.experimental.pallas import tpu as pltpu
from jax.experimental.pallas import tpu_sc as plsc

N = 10000
E = 320000
D = 128
DE = 16
ENF = 64

_INFO = plsc.get_sparse_core_info()
NC = _INFO.num_cores
NS = _INFO.num_subcores
NW = NC * NS
EPW = E // NW
CH = 80
HCH = CH // 2
NITER = EPW // CH
RPT = 624
RPT_LAST = N - 15 * RPT

_mesh = plsc.VectorSubcoreMesh(core_axis_name="c", subcore_axis_name="s")

_PERM = [32 * k + 2 * j + r for k in range(2) for r in range(2)
         for j in range(16)]


def _proj_body(h_ref, w_ref, hst_ref):
    hst_ref[...] = jnp.dot(h_ref[...], w_ref[...],
                           preferred_element_type=jnp.float32)


def _project(h, wst):
    return pl.pallas_call(
        _proj_body,
        out_shape=jax.ShapeDtypeStruct((N, D), jnp.float32),
    )(h, wst)


@functools.partial(
    pl.kernel,
    mesh=_mesh,
    out_type=jax.ShapeDtypeStruct((E // 2, D), jnp.float32),
    scratch_types=[
        pltpu.VMEM((NITER, CH), jnp.int32),
        pltpu.VMEM((NITER, CH), jnp.int32),
        pltpu.VMEM((3, CH, D), jnp.float32),
        pltpu.VMEM((3, CH, D), jnp.float32),
        pltpu.VMEM((2, HCH, D), jnp.float32),
        pltpu.SemaphoreType.DMA((3,)),
        pltpu.SemaphoreType.DMA((3,)),
        pltpu.SemaphoreType.DMA((2,)),
    ],
)
def _gather_k(hst_hbm, row_hbm, col_hbm, u_hbm,
              idxr, idxc, g1, g2, ub, sem1, sem2, semu):
    wid = lax.axis_index("s") * NC + lax.axis_index("c")
    pltpu.sync_copy(row_hbm.at[wid], idxr)
    pltpu.sync_copy(col_hbm.at[wid], idxc)
    ubase = wid * (EPW // 2)

    def start(i, s):
        pltpu.async_copy(hst_hbm.at[idxr.at[i]], g1.at[s], sem1.at[s])
        pltpu.async_copy(hst_hbm.at[idxc.at[i]], g2.at[s], sem2.at[s])

    def wait(s):
        pltpu.make_async_copy(hst_hbm.at[pl.ds(0, CH)], g1.at[s],
                              sem1.at[s]).wait()
        pltpu.make_async_copy(hst_hbm.at[pl.ds(0, CH)], g2.at[s],
                              sem2.at[s]).wait()

    start(0, 0)
    start(1, 1)

    def body(i, carry):
        s = lax.rem(i, 3)
        su = i & 1

        @pl.when(i + 2 < NITER)
        def _():
            start(i + 2, lax.rem(i + 2, 3))

        wait(s)

        @pl.when(i >= 2)
        def _():
            pltpu.make_async_copy(u_hbm.at[pl.ds(0, HCH)], ub.at[su],
                                  semu.at[su]).wait()

        def cbody(q, c2):
            for dp in range(4):
                p = 4 * q + dp
                e0 = 2 * p
                e1 = e0 + 1
                for k in range(4):
                    lo = 16 * k
                    hi = ENF + lo
                    ub[su, p, pl.ds(lo, 16)] = (g1[s, e0, pl.ds(lo, 16)]
                                                + g2[s, e0, pl.ds(hi, 16)])
                    ub[su, p, pl.ds(hi, 16)] = (g1[s, e1, pl.ds(lo, 16)]
                                                + g2[s, e1, pl.ds(hi, 16)])
            return c2

        lax.fori_loop(0, HCH // 4, cbody, 0)
        uoff = pl.multiple_of(ubase + i * HCH, HCH)
        pltpu.async_copy(ub.at[su], u_hbm.at[pl.ds(uoff, HCH)], semu.at[su])
        return carry

    lax.fori_loop(0, NITER, body, 0)
    pltpu.make_async_copy(u_hbm.at[pl.ds(0, HCH)], ub.at[0], semu.at[0]).wait()
    pltpu.make_async_copy(u_hbm.at[pl.ds(0, HCH)], ub.at[1], semu.at[1]).wait()


def _edge_mlp_body(u_ref, ea_ref, wa_ref, b1_ref, w2_ref, b2_ref, out_ref):
    u = u_ref[...]
    ea = ea_ref[...]

    def half(ux, eax):
        pre = (ux
               + jnp.dot(eax, wa_ref[...], preferred_element_type=jnp.float32)
               + b1_ref[...])
        t = jax.nn.silu(pre)
        ef = jax.nn.silu(
            jnp.dot(t, w2_ref[...], preferred_element_type=jnp.float32)
            + b2_ref[...])
        return jnp.concatenate([ef, jnp.zeros_like(ef)], axis=1)

    out_ref[...] = jnp.concatenate(
        [half(u[:, :ENF], ea[:, :DE]), half(u[:, ENF:], ea[:, DE:])], axis=0)


def _edge_mlp(u2, ea2, wa, b1, w2, b2):
    BEH = 2000
    grid = (E // 2 // BEH,)
    blk = lambda r, c: pl.BlockSpec((r, c), lambda i: (i, 0))
    fixed = lambda r, c: pl.BlockSpec((r, c), lambda i: (0, 0))
    return pl.pallas_call(
        _edge_mlp_body,
        grid=grid,
        in_specs=[blk(BEH, D), blk(BEH, 2 * DE),
                  fixed(DE, ENF), fixed(1, ENF), fixed(ENF, ENF),
                  fixed(1, ENF)],
        out_specs=blk(2 * BEH, D),
        out_shape=jax.ShapeDtypeStruct((E, D), jnp.float32),
    )(u2, ea2, wa, b1, w2, b2)


@functools.partial(
    pl.kernel,
    mesh=_mesh,
    out_type=jax.ShapeDtypeStruct((NC, N, D), jnp.float32),
    scratch_types=[
        pltpu.VMEM((NITER, CH), jnp.int32),
        pltpu.VMEM((2, CH, D), jnp.float32),
        pltpu.VMEM_SHARED((N, D), jnp.float32),
        pltpu.SemaphoreType.DMA((2,)),
    ],
)
def _scatter_k(ef_hbm, row_hbm, zero_hbm, out_hbm, idxd, buf, acc, sem):
    cid = lax.axis_index("c")
    sid = lax.axis_index("s")
    wid = sid * NC + cid
    start = pl.multiple_of(sid * RPT, 8)
    @pl.when(sid < NS - 1)
    def _():
        pltpu.sync_copy(zero_hbm.at[pl.ds(start, RPT)],
                        acc.at[pl.ds(start, RPT)])

    @pl.when(sid == NS - 1)
    def _():
        pltpu.sync_copy(zero_hbm.at[pl.ds(start, RPT_LAST)],
                        acc.at[pl.ds(start, RPT_LAST)])

    pltpu.sync_copy(row_hbm.at[wid], idxd)
    plsc.subcore_barrier()
    base = wid * EPW

    def load(i, s):
        off = pl.multiple_of(base + i * CH, CH)
        pltpu.async_copy(ef_hbm.at[pl.ds(off, CH)], buf.at[s], sem.at[s])

    def wait(s):
        pltpu.make_async_copy(ef_hbm.at[pl.ds(0, CH)], buf.at[s],
                              sem.at[s]).wait()

    load(0, 0)

    def body(i, carry):
        s = i & 1

        @pl.when(i + 1 < NITER)
        def _():
            load(i + 1, 1 - s)

        wait(s)
        pltpu.sync_copy(buf.at[s], acc.at[idxd.at[i]], add=True)
        return carry

    lax.fori_loop(0, NITER, body, 0)
    plsc.subcore_barrier()

    @pl.when(sid < NS - 1)
    def _():
        pltpu.sync_copy(acc.at[pl.ds(start, RPT)],
                        out_hbm.at[cid, pl.ds(start, RPT)])

    @pl.when(sid == NS - 1)
    def _():
        pltpu.sync_copy(acc.at[pl.ds(start, RPT_LAST)],
                        out_hbm.at[cid, pl.ds(start, RPT_LAST)])


def _node_mlp_body(h_ref, a0_ref, a1_ref, wh_ref, wa_ref, b1_ref, w2_ref,
                   b2_ref, out_ref):
    hcur = h_ref[...]
    agg = a0_ref[:, :ENF] + a1_ref[:, :ENF]
    z = jax.nn.silu(
        jnp.dot(hcur, wh_ref[...], preferred_element_type=jnp.float32)
        + jnp.dot(agg, wa_ref[...], preferred_element_type=jnp.float32)
        + b1_ref[...])
    out_ref[...] = (hcur
                    + jnp.dot(z, w2_ref[...],
                              preferred_element_type=jnp.float32)
                    + b2_ref[...])


def _node_mlp(h, a0, a1, wh, wa, b1, w2, b2):
    return pl.pallas_call(
        _node_mlp_body,
        out_shape=jax.ShapeDtypeStruct((N, D), jnp.float32),
    )(h, a0, a1, wh, wa, b1, w2, b2)


def kernel(h, edge_index, edge_attr, We1, be1, We2, be2, Wn1, bn1, Wn2, bn2):
    row = edge_index[0].astype(jnp.int32)
    col = edge_index[1].astype(jnp.int32)
    wst = jnp.concatenate([We1[:D], We1[D:2 * D]], axis=1)
    hst = _project(h, wst)
    row3 = row.reshape(NW, NITER, CH)
    col3 = col.reshape(NW, NITER, CH)
    u2 = _gather_k(hst, row3, col3)
    ea2 = edge_attr.reshape(E // 2, 2 * DE)
    ef = _edge_mlp(u2, ea2, We1[2 * D:], be1.reshape(1, ENF), We2,
                   be2.reshape(1, ENF))
    rowp = row.reshape(E // 4000, 2000, 2).transpose(0, 2, 1).reshape(
        NW, NITER, CH)
    parts = _scatter_k(ef, rowp, jnp.zeros((N, D), jnp.float32))
    return _node_mlp(h, parts[0], parts[1], Wn1[:D], Wn1[D:],
                     bn1.reshape(1, D), Wn2, bn2.reshape(1, D))


# speedup vs baseline: 1.0484x; 1.0041x over previous
import functools

import jax
import jax.numpy as jnp
from jax import lax
from jax.experimental import pallas as pl
from jax.experimental.pallas import tpu as pltpu
from jax.experimental.pallas import tpu_sc as plsc

N = 10000
E = 320000
D = 128
DE = 16
ENF = 64

_INFO = plsc.get_sparse_core_info()
NC = _INFO.num_cores
NS = _INFO.num_subcores
NW = NC * NS
EPW = E // NW
CH = 80
HCH = CH // 2
NITER = EPW // CH
RPT = 624
RPT_LAST = N - 15 * RPT

_mesh = plsc.VectorSubcoreMesh(core_axis_name="c", subcore_axis_name="s")

_PERM = [32 * k + 2 * j + r for k in range(2) for r in range(2)
         for j in range(16)]


def _proj_body(h_ref, w_ref, hst_ref):
    hst_ref[...] = jnp.dot(h_ref[...], w_ref[...],
                           preferred_element_type=jnp.float32)


def _project(h, wst):
    return pl.pallas_call(
        _proj_body,
        out_shape=jax.ShapeDtypeStruct((N, D), jnp.float32),
    )(h, wst)


@functools.partial(
    pl.kernel,
    mesh=_mesh,
    out_type=jax.ShapeDtypeStruct((E // 2, D), jnp.float32),
    scratch_types=[
        pltpu.VMEM((EPW,), jnp.int32),
        pltpu.VMEM((EPW,), jnp.int32),
        pltpu.VMEM((3, CH, D), jnp.float32),
        pltpu.VMEM((3, CH, D), jnp.float32),
        pltpu.VMEM((2, HCH, D), jnp.float32),
        pltpu.SemaphoreType.DMA((3,)),
        pltpu.SemaphoreType.DMA((3,)),
        pltpu.SemaphoreType.DMA((2,)),
    ],
)
def _gather_k(hst_hbm, row_hbm, col_hbm, u_hbm,
              idxr, idxc, g1, g2, ub, sem1, sem2, semu):
    wid = lax.axis_index("s") * NC + lax.axis_index("c")
    ebase = pl.multiple_of(wid * EPW, 8)
    pltpu.sync_copy(row_hbm.at[pl.ds(ebase, EPW)], idxr)
    pltpu.sync_copy(col_hbm.at[pl.ds(ebase, EPW)], idxc)
    ubase = wid * (EPW // 2)

    def start(i, s):
        pltpu.async_copy(hst_hbm.at[idxr.at[pl.ds(i * CH, CH)]],
                         g1.at[s], sem1.at[s])
        pltpu.async_copy(hst_hbm.at[idxc.at[pl.ds(i * CH, CH)]],
                         g2.at[s], sem2.at[s])

    def wait(s):
        pltpu.make_async_copy(hst_hbm.at[pl.ds(0, CH)], g1.at[s],
                              sem1.at[s]).wait()
        pltpu.make_async_copy(hst_hbm.at[pl.ds(0, CH)], g2.at[s],
                              sem2.at[s]).wait()

    start(0, 0)
    start(1, 1)

    def body(i, carry):
        s = lax.rem(i, 3)
        su = i & 1

        @pl.when(i + 2 < NITER)
        def _():
            start(i + 2, lax.rem(i + 2, 3))

        wait(s)

        @pl.when(i >= 2)
        def _():
            pltpu.make_async_copy(u_hbm.at[pl.ds(0, HCH)], ub.at[su],
                                  semu.at[su]).wait()

        def cbody(q, c2):
            for dp in range(4):
                p = 4 * q + dp
                e0 = 2 * p
                e1 = e0 + 1
                for k in range(4):
                    lo = 16 * k
                    hi = ENF + lo
                    ub[su, p, pl.ds(lo, 16)] = (g1[s, e0, pl.ds(lo, 16)]
                                                + g2[s, e0, pl.ds(hi, 16)])
                    ub[su, p, pl.ds(hi, 16)] = (g1[s, e1, pl.ds(lo, 16)]
                                                + g2[s, e1, pl.ds(hi, 16)])
            return c2

        lax.fori_loop(0, HCH // 4, cbody, 0)
        uoff = pl.multiple_of(ubase + i * HCH, HCH)
        pltpu.async_copy(ub.at[su], u_hbm.at[pl.ds(uoff, HCH)], semu.at[su])
        return carry

    lax.fori_loop(0, NITER, body, 0)
    pltpu.make_async_copy(u_hbm.at[pl.ds(0, HCH)], ub.at[0], semu.at[0]).wait()
    pltpu.make_async_copy(u_hbm.at[pl.ds(0, HCH)], ub.at[1], semu.at[1]).wait()


def _edge_mlp_body(u_ref, ea_ref, wa_ref, b1_ref, w2_ref, b2_ref, out_ref):
    u = u_ref[...]
    ea = ea_ref[...]

    def half(ux, eax):
        pre = (ux
               + jnp.dot(eax, wa_ref[...], preferred_element_type=jnp.float32)
               + b1_ref[...])
        t = jax.nn.silu(pre)
        ef = jax.nn.silu(
            jnp.dot(t, w2_ref[...], preferred_element_type=jnp.float32)
            + b2_ref[...])
        return jnp.concatenate([ef, jnp.zeros_like(ef)], axis=1)

    out_ref[...] = jnp.concatenate(
        [half(u[:, :ENF], ea[:, :DE]), half(u[:, ENF:], ea[:, DE:])], axis=0)


def _edge_mlp(u2, ea2, wa, b1, w2, b2):
    BEH = 2000
    grid = (E // 2 // BEH,)
    blk = lambda r, c: pl.BlockSpec((r, c), lambda i: (i, 0))
    fixed = lambda r, c: pl.BlockSpec((r, c), lambda i: (0, 0))
    return pl.pallas_call(
        _edge_mlp_body,
        grid=grid,
        in_specs=[blk(BEH, D), blk(BEH, 2 * DE),
                  fixed(DE, ENF), fixed(1, ENF), fixed(ENF, ENF),
                  fixed(1, ENF)],
        out_specs=blk(2 * BEH, D),
        out_shape=jax.ShapeDtypeStruct((E, D), jnp.float32),
    )(u2, ea2, wa, b1, w2, b2)


@functools.partial(
    pl.kernel,
    mesh=_mesh,
    out_type=jax.ShapeDtypeStruct((NC, N, D), jnp.float32),
    scratch_types=[
        pltpu.VMEM((NITER, CH), jnp.int32),
        pltpu.VMEM((2, CH, D), jnp.float32),
        pltpu.VMEM_SHARED((N, D), jnp.float32),
        pltpu.SemaphoreType.DMA((2,)),
    ],
)
def _scatter_k(ef_hbm, row_hbm, zero_hbm, out_hbm, idxd, buf, acc, sem):
    cid = lax.axis_index("c")
    sid = lax.axis_index("s")
    wid = sid * NC + cid
    start = pl.multiple_of(sid * RPT, 8)
    @pl.when(sid < NS - 1)
    def _():
        pltpu.sync_copy(zero_hbm.at[pl.ds(start, RPT)],
                        acc.at[pl.ds(start, RPT)])

    @pl.when(sid == NS - 1)
    def _():
        pltpu.sync_copy(zero_hbm.at[pl.ds(start, RPT_LAST)],
                        acc.at[pl.ds(start, RPT_LAST)])

    pltpu.sync_copy(row_hbm.at[wid], idxd)
    plsc.subcore_barrier()
    base = wid * EPW

    def load(i, s):
        off = pl.multiple_of(base + i * CH, CH)
        pltpu.async_copy(ef_hbm.at[pl.ds(off, CH)], buf.at[s], sem.at[s])

    def wait(s):
        pltpu.make_async_copy(ef_hbm.at[pl.ds(0, CH)], buf.at[s],
                              sem.at[s]).wait()

    load(0, 0)

    def body(i, carry):
        s = i & 1

        @pl.when(i + 1 < NITER)
        def _():
            load(i + 1, 1 - s)

        wait(s)
        pltpu.sync_copy(buf.at[s], acc.at[idxd.at[i]], add=True)
        return carry

    lax.fori_loop(0, NITER, body, 0)
    plsc.subcore_barrier()

    @pl.when(sid < NS - 1)
    def _():
        pltpu.sync_copy(acc.at[pl.ds(start, RPT)],
                        out_hbm.at[cid, pl.ds(start, RPT)])

    @pl.when(sid == NS - 1)
    def _():
        pltpu.sync_copy(acc.at[pl.ds(start, RPT_LAST)],
                        out_hbm.at[cid, pl.ds(start, RPT_LAST)])


def _node_mlp_body(h_ref, a0_ref, a1_ref, wh_ref, wa_ref, b1_ref, w2_ref,
                   b2_ref, out_ref):
    hcur = h_ref[...]
    agg = a0_ref[:, :ENF] + a1_ref[:, :ENF]
    z = jax.nn.silu(
        jnp.dot(hcur, wh_ref[...], preferred_element_type=jnp.float32)
        + jnp.dot(agg, wa_ref[...], preferred_element_type=jnp.float32)
        + b1_ref[...])
    out_ref[...] = (hcur
                    + jnp.dot(z, w2_ref[...],
                              preferred_element_type=jnp.float32)
                    + b2_ref[...])


def _node_mlp(h, a0, a1, wh, wa, b1, w2, b2):
    return pl.pallas_call(
        _node_mlp_body,
        out_shape=jax.ShapeDtypeStruct((N, D), jnp.float32),
    )(h, a0, a1, wh, wa, b1, w2, b2)


def kernel(h, edge_index, edge_attr, We1, be1, We2, be2, Wn1, bn1, Wn2, bn2):
    row = edge_index[0].astype(jnp.int32)
    col = edge_index[1].astype(jnp.int32)
    wst = jnp.concatenate([We1[:D], We1[D:2 * D]], axis=1)
    hst = _project(h, wst)
    u2 = _gather_k(hst, row, col)
    ea2 = edge_attr.reshape(E // 2, 2 * DE)
    ef = _edge_mlp(u2, ea2, We1[2 * D:], be1.reshape(1, ENF), We2,
                   be2.reshape(1, ENF))
    rowp = row.reshape(E // 4000, 2000, 2).transpose(0, 2, 1).reshape(
        NW, NITER, CH)
    parts = _scatter_k(ef, rowp, jnp.zeros((N, D), jnp.float32))
    return _node_mlp(h, parts[0], parts[1], Wn1[:D], Wn1[D:],
                     bn1.reshape(1, D), Wn2, bn2.reshape(1, D))


# speedup vs baseline: 1.0493x; 1.0009x over previous
import functools

import jax
import jax.numpy as jnp
from jax import lax
from jax.experimental import pallas as pl
from jax.experimental.pallas import tpu as pltpu
from jax.experimental.pallas import tpu_sc as plsc

N = 10000
E = 320000
D = 128
DE = 16
ENF = 64

_INFO = plsc.get_sparse_core_info()
NC = _INFO.num_cores
NS = _INFO.num_subcores
NW = NC * NS
EPW = E // NW
CH = 80
HCH = CH // 2
NITER = EPW // CH
RPT = 624
RPT_LAST = N - 15 * RPT

_mesh = plsc.VectorSubcoreMesh(core_axis_name="c", subcore_axis_name="s")


def _proj_body(h_ref, w_ref, hst_ref):
    hst_ref[...] = jnp.dot(h_ref[...], w_ref[...],
                           preferred_element_type=jnp.float32)


def _project(h, wst):
    return pl.pallas_call(
        _proj_body,
        out_shape=jax.ShapeDtypeStruct((N, D), jnp.float32),
    )(h, wst)


@functools.partial(
    pl.kernel,
    mesh=_mesh,
    out_type=jax.ShapeDtypeStruct((E // 2, D), jnp.float32),
    scratch_types=[
        pltpu.VMEM((EPW,), jnp.int32),
        pltpu.VMEM((EPW,), jnp.int32),
        pltpu.VMEM((3, CH, D), jnp.float32),
        pltpu.VMEM((3, CH, D), jnp.float32),
        pltpu.VMEM((2, HCH, D), jnp.float32),
        pltpu.SemaphoreType.DMA((3,)),
        pltpu.SemaphoreType.DMA((3,)),
        pltpu.SemaphoreType.DMA((2,)),
    ],
)
def _gather_k(hst_hbm, row_hbm, col_hbm, u_hbm,
              idxr, idxc, g1, g2, ub, sem1, sem2, semu):
    wid = lax.axis_index("s") * NC + lax.axis_index("c")
    ebase = pl.multiple_of(wid * EPW, 8)
    pltpu.sync_copy(row_hbm.at[pl.ds(ebase, EPW)], idxr)
    pltpu.sync_copy(col_hbm.at[pl.ds(ebase, EPW)], idxc)
    ubase = wid * (EPW // 2)

    def start(i, s):
        pltpu.async_copy(hst_hbm.at[idxr.at[pl.ds(i * CH, CH)]],
                         g1.at[s], sem1.at[s])
        pltpu.async_copy(hst_hbm.at[idxc.at[pl.ds(i * CH, CH)]],
                         g2.at[s], sem2.at[s])

    def wait(s):
        pltpu.make_async_copy(hst_hbm.at[pl.ds(0, CH)], g1.at[s],
                              sem1.at[s]).wait()
        pltpu.make_async_copy(hst_hbm.at[pl.ds(0, CH)], g2.at[s],
                              sem2.at[s]).wait()

    start(0, 0)
    start(1, 1)

    def body(i, carry):
        s = lax.rem(i, 3)
        su = i & 1

        @pl.when(i + 2 < NITER)
        def _():
            start(i + 2, lax.rem(i + 2, 3))

        wait(s)

        @pl.when(i >= 2)
        def _():
            pltpu.make_async_copy(u_hbm.at[pl.ds(0, HCH)], ub.at[su],
                                  semu.at[su]).wait()

        def cbody(q, c2):
            for dp in range(4):
                p = 4 * q + dp
                e0 = 2 * p
                e1 = e0 + 1
                for k in range(4):
                    lo = 16 * k
                    hi = ENF + lo
                    ub[su, p, pl.ds(lo, 16)] = (g1[s, e0, pl.ds(lo, 16)]
                                                + g2[s, e0, pl.ds(hi, 16)])
                    ub[su, p, pl.ds(hi, 16)] = (g1[s, e1, pl.ds(lo, 16)]
                                                + g2[s, e1, pl.ds(hi, 16)])
            return c2

        lax.fori_loop(0, HCH // 4, cbody, 0)
        uoff = pl.multiple_of(ubase + i * HCH, HCH)
        pltpu.async_copy(ub.at[su], u_hbm.at[pl.ds(uoff, HCH)], semu.at[su])
        return carry

    lax.fori_loop(0, NITER, body, 0)
    pltpu.make_async_copy(u_hbm.at[pl.ds(0, HCH)], ub.at[0], semu.at[0]).wait()
    pltpu.make_async_copy(u_hbm.at[pl.ds(0, HCH)], ub.at[1], semu.at[1]).wait()


def _edge_mlp_body(u_ref, ea_ref, wa_ref, b1_ref, w2_ref, b2_ref, out_ref):
    u = u_ref[...]
    ea = ea_ref[...]

    def half(ux, eax):
        pre = (ux
               + jnp.dot(eax, wa_ref[...], preferred_element_type=jnp.float32)
               + b1_ref[...])
        t = jax.nn.silu(pre)
        ef = jax.nn.silu(
            jnp.dot(t, w2_ref[...], preferred_element_type=jnp.float32)
            + b2_ref[...])
        return jnp.concatenate([ef, jnp.zeros_like(ef)], axis=1)

    out_ref[...] = jnp.concatenate(
        [half(u[:, :ENF], ea[:, :DE]), half(u[:, ENF:], ea[:, DE:])], axis=0)


def _edge_mlp(u2, ea2, wa, b1, w2, b2):
    BEH = 2000
    grid = (E // 2 // BEH,)
    blk = lambda r, c: pl.BlockSpec((r, c), lambda i: (i, 0))
    fixed = lambda r, c: pl.BlockSpec((r, c), lambda i: (0, 0))
    return pl.pallas_call(
        _edge_mlp_body,
        grid=grid,
        in_specs=[blk(BEH, D), blk(BEH, 2 * DE),
                  fixed(DE, ENF), fixed(1, ENF), fixed(ENF, ENF),
                  fixed(1, ENF)],
        out_specs=blk(2 * BEH, D),
        out_shape=jax.ShapeDtypeStruct((E, D), jnp.float32),
    )(u2, ea2, wa, b1, w2, b2)


@functools.partial(
    pl.kernel,
    mesh=_mesh,
    out_type=jax.ShapeDtypeStruct((NC, N, D), jnp.float32),
    scratch_types=[
        pltpu.VMEM((NITER, CH), jnp.int32),
        pltpu.VMEM((2, CH, D), jnp.float32),
        pltpu.VMEM_SHARED((N, D), jnp.float32),
        pltpu.SemaphoreType.DMA((2,)),
    ],
)
def _scatter_k(ef_hbm, row_hbm, zero_hbm, out_hbm, idxd, buf, acc, sem):
    cid = lax.axis_index("c")
    sid = lax.axis_index("s")
    wid = sid * NC + cid
    start = pl.multiple_of(sid * RPT, 8)
    @pl.when(sid < NS - 1)
    def _():
        pltpu.sync_copy(zero_hbm.at[pl.ds(start, RPT)],
                        acc.at[pl.ds(start, RPT)])

    @pl.when(sid == NS - 1)
    def _():
        pltpu.sync_copy(zero_hbm.at[pl.ds(start, RPT_LAST)],
                        acc.at[pl.ds(start, RPT_LAST)])

    pltpu.sync_copy(row_hbm.at[wid], idxd)
    plsc.subcore_barrier()
    base = wid * EPW

    def load(i, s):
        off = pl.multiple_of(base + i * CH, CH)
        pltpu.async_copy(ef_hbm.at[pl.ds(off, CH)], buf.at[s], sem.at[s])

    def wait(s):
        pltpu.make_async_copy(ef_hbm.at[pl.ds(0, CH)], buf.at[s],
                              sem.at[s]).wait()

    load(0, 0)

    def body(i, carry):
        s = i & 1

        @pl.when(i + 1 < NITER)
        def _():
            load(i + 1, 1 - s)

        wait(s)
        pltpu.sync_copy(buf.at[s], acc.at[idxd.at[i]], add=True)
        return carry

    lax.fori_loop(0, NITER, body, 0)
    plsc.subcore_barrier()

    @pl.when(sid < NS - 1)
    def _():
        pltpu.sync_copy(acc.at[pl.ds(start, RPT)],
                        out_hbm.at[cid, pl.ds(start, RPT)])

    @pl.when(sid == NS - 1)
    def _():
        pltpu.sync_copy(acc.at[pl.ds(start, RPT_LAST)],
                        out_hbm.at[cid, pl.ds(start, RPT_LAST)])


def _node_mlp_body(h_ref, a0_ref, a1_ref, wh_ref, wa_ref, b1_ref, w2_ref,
                   b2_ref, out_ref):
    hcur = h_ref[...]
    agg = a0_ref[:, :ENF] + a1_ref[:, :ENF]
    z = jax.nn.silu(
        jnp.dot(hcur, wh_ref[...], preferred_element_type=jnp.float32)
        + jnp.dot(agg, wa_ref[...], preferred_element_type=jnp.float32)
        + b1_ref[...])
    out_ref[...] = (hcur
                    + jnp.dot(z, w2_ref[...],
                              preferred_element_type=jnp.float32)
                    + b2_ref[...])


def _node_mlp(h, a0, a1, wh, wa, b1, w2, b2):
    return pl.pallas_call(
        _node_mlp_body,
        out_shape=jax.ShapeDtypeStruct((N, D), jnp.float32),
    )(h, a0, a1, wh, wa, b1, w2, b2)


def kernel(h, edge_index, edge_attr, We1, be1, We2, be2, Wn1, bn1, Wn2, bn2):
    row = edge_index[0].astype(jnp.int32)
    col = edge_index[1].astype(jnp.int32)
    wst = jnp.concatenate([We1[:D], We1[D:2 * D]], axis=1)
    hst = _project(h, wst)
    u2 = _gather_k(hst, row, col)
    ea2 = edge_attr.reshape(E // 2, 2 * DE)
    ef = _edge_mlp(u2, ea2, We1[2 * D:], be1.reshape(1, ENF), We2,
                   be2.reshape(1, ENF))
    rowp = row.reshape(E // 4000, 2000, 2).transpose(0, 2, 1).reshape(
        NW, NITER, CH)
    parts = _scatter_k(ef, rowp, jnp.zeros((N, D), jnp.float32))
    return _node_mlp(h, parts[0], parts[1], Wn1[:D], Wn1[D:],
                     bn1.reshape(1, D), Wn2, bn2.reshape(1, D))
